# Initial kernel scaffold; baseline (speedup 1.0000x reference)
import jax
import jax.numpy as jnp
from jax.experimental import pallas as pl


def kernel(x, edge_index, edge_attr, batch, W_lin, b_lin, gg_weight, gru_wih, gru_whh, gru_bih, gru_bhh, W_cls, b_cls):
    raise NotImplementedError("write your pallas kernel here")


# speedup vs baseline: 10.9560x; 10.9560x over previous
import functools

import jax
import jax.numpy as jnp
from jax import lax
from jax.experimental import pallas as pl
from jax.experimental.pallas import tpu as pltpu
from jax.experimental.pallas import tpu_sc as plsc

N_NODES = 10000
N_PAD = 10240
D = 128
N_TYPES = 13
N_STEPS = 8
N_GRAPHS = 64
N_EDGES = 320000
CHUNK = 128
C_TOT = N_EDGES // CHUNK + N_TYPES
RB = 512
NRB = N_PAD // RB
TRASH = N_NODES
ROWS_PER_TILE = N_PAD // 16
F32 = jnp.float32


def _sc_scatter_body(mflat, ed, meta, zeros, out, agg_sh, idx_v, rows_v,
                     meta_vm, sem):
  core = lax.axis_index("c")
  sub = lax.axis_index("s")
  pltpu.sync_copy(meta, meta_vm)
  for t in range(N_TYPES):
    @pl.when(core == (t % 2))
    def _process_type():
      for z in range(ROWS_PER_TILE // CHUNK):
        pltpu.sync_copy(
            zeros, agg_sh.at[pl.ds(sub * ROWS_PER_TILE + z * CHUNK, CHUNK)])
      plsc.subcore_barrier()
      start = jnp.max(meta_vm[2 * t])
      nck = jnp.max(meta_vm[2 * t + 1])

      @pl.loop(sub, nck, step=16)
      def _chunk(j):
        pltpu.sync_copy(ed.at[start + j], idx_v)
        pltpu.async_copy(mflat.at[idx_v.at[0]], rows_v, sem).wait()
        pltpu.sync_copy(rows_v, agg_sh.at[idx_v.at[1]], add=True)

      plsc.subcore_barrier()
      pltpu.sync_copy(
          agg_sh.at[pl.ds(sub * ROWS_PER_TILE, ROWS_PER_TILE)],
          out.at[pl.ds(t * N_PAD + sub * ROWS_PER_TILE, ROWS_PER_TILE)])


def _make_sc_scatter():
  mesh = plsc.VectorSubcoreMesh(core_axis_name="c", subcore_axis_name="s")
  return pl.kernel(
      _sc_scatter_body,
      out_type=jax.ShapeDtypeStruct((N_TYPES * N_PAD, D), F32),
      mesh=mesh,
      scratch_types=[
          pltpu.VMEM_SHARED((N_PAD, D), F32),
          pltpu.VMEM((2, CHUNK), jnp.int32),
          pltpu.VMEM((CHUNK, D), F32),
          pltpu.VMEM((32, 16), jnp.int32),
          pltpu.SemaphoreType.DMA,
      ],
      compiler_params=pltpu.CompilerParams(needs_layout_passes=False),
  )


def _init_body(x_ref, wlt_ref, bl_ref, w0_ref, h_ref, m_ref):
  h0 = jnp.dot(x_ref[...], wlt_ref[...],
               preferred_element_type=F32) + bl_ref[...]
  h_ref[0] = h0
  m_ref[0] = jnp.dot(h0, w0_ref[0], preferred_element_type=F32)


def _gru_core(agg_ref, h_ref, wih_ref, whh_ref, bih_ref, bhh_ref):
  h = h_ref[0]
  gi = jnp.dot(agg_ref[0], wih_ref[0], preferred_element_type=F32) + bih_ref[0]
  gh = jnp.dot(h, whh_ref[0], preferred_element_type=F32) + bhh_ref[0]
  r = jax.nn.sigmoid(gi[:, :D] + gh[:, :D])
  z = jax.nn.sigmoid(gi[:, D:2 * D] + gh[:, D:2 * D])
  n = jnp.tanh(gi[:, 2 * D:] + r * gh[:, 2 * D:])
  return (1.0 - z) * n + z * h


def _step_body(agg_ref, h_ref, wih_ref, whh_ref, bih_ref, bhh_ref, wn_ref,
               hn_ref, mn_ref):
  hn = _gru_core(agg_ref, h_ref, wih_ref, whh_ref, bih_ref, bhh_ref)
  hn_ref[0] = hn
  mn_ref[0] = jnp.dot(hn, wn_ref[0], preferred_element_type=F32)


def _last_step_body(agg_ref, h_ref, wih_ref, whh_ref, bih_ref, bhh_ref,
                    hn_ref):
  hn_ref[0] = _gru_core(agg_ref, h_ref, wih_ref, whh_ref, bih_ref, bhh_ref)


def _pool_body(h_ref, valid_ref, batch_ref, wcls_ref, bcls_ref, out_ref,
               sums_ref, cnts_ref):
  rb = pl.program_id(0)

  @pl.when(rb == 0)
  def _():
    sums_ref[...] = jnp.zeros_like(sums_ref)
    cnts_ref[...] = jnp.zeros_like(cnts_ref)

  msg = jnp.zeros((RB, D), F32)
  for t in range(N_TYPES):
    msg = msg + h_ref[t] * valid_ref[t]
  b = batch_ref[0]
  gid = lax.broadcasted_iota(jnp.int32, (N_GRAPHS, RB), 0)
  onehot = (gid == jnp.broadcast_to(b, (N_GRAPHS, RB))).astype(F32)
  sums_ref[...] += jnp.dot(onehot, msg, preferred_element_type=F32)
  cnts_ref[...] += jnp.dot(onehot, jnp.ones((RB, D), F32),
                           preferred_element_type=F32)

  @pl.when(rb == NRB - 1)
  def _():
    pooled = sums_ref[...] / jnp.maximum(cnts_ref[...], 1.0)
    out_ref[...] = jnp.dot(pooled, wcls_ref[...],
                           preferred_element_type=F32) + bcls_ref[...]


def _tc_specs():
  blk_h = pl.BlockSpec((1, RB, D), lambda t, rb: (t, rb, 0))
  blk_w = pl.BlockSpec((1, D, 3 * D), lambda t, rb: (t, 0, 0))
  blk_b = pl.BlockSpec((1, 1, 3 * D), lambda t, rb: (t, 0, 0))
  blk_wn = pl.BlockSpec((1, D, D), lambda t, rb: (t, 0, 0))
  return blk_h, blk_w, blk_b, blk_wn


def _init_call(xpad, wlinT, bl2, w0):
  blk_h, _, _, blk_wn = _tc_specs()
  return pl.pallas_call(
      _init_body,
      grid=(N_TYPES, NRB),
      in_specs=[
          pl.BlockSpec((RB, D), lambda t, rb: (rb, 0)),
          pl.BlockSpec((D, D), lambda t, rb: (0, 0)),
          pl.BlockSpec((1, D), lambda t, rb: (0, 0)),
          blk_wn,
      ],
      out_specs=[blk_h, blk_h],
      out_shape=[
          jax.ShapeDtypeStruct((N_TYPES, N_PAD, D), F32),
          jax.ShapeDtypeStruct((N_TYPES, N_PAD, D), F32),
      ],
  )(xpad, wlinT, bl2, w0)


def _step_call(agg, h, wihT, whhT, bih3, bhh3, wn):
  blk_h, blk_w, blk_b, blk_wn = _tc_specs()
  return pl.pallas_call(
      _step_body,
      grid=(N_TYPES, NRB),
      in_specs=[blk_h, blk_h, blk_w, blk_w, blk_b, blk_b, blk_wn],
      out_specs=[blk_h, blk_h],
      out_shape=[
          jax.ShapeDtypeStruct((N_TYPES, N_PAD, D), F32),
          jax.ShapeDtypeStruct((N_TYPES, N_PAD, D), F32),
      ],
  )(agg, h, wihT, whhT, bih3, bhh3, wn)


def _last_step_call(agg, h, wihT, whhT, bih3, bhh3):
  blk_h, blk_w, blk_b, _ = _tc_specs()
  return pl.pallas_call(
      _last_step_body,
      grid=(N_TYPES, NRB),
      in_specs=[blk_h, blk_h, blk_w, blk_w, blk_b, blk_b],
      out_specs=blk_h,
      out_shape=jax.ShapeDtypeStruct((N_TYPES, N_PAD, D), F32),
  )(agg, h, wihT, whhT, bih3, bhh3)


def _pool_call(hfin, valid3, batch3, wclsT, bcls2):
  return pl.pallas_call(
      _pool_body,
      grid=(NRB,),
      in_specs=[
          pl.BlockSpec((N_TYPES, RB, D), lambda rb: (0, rb, 0)),
          pl.BlockSpec((N_TYPES, 1, D), lambda rb: (0, 0, 0)),
          pl.BlockSpec((1, 1, RB), lambda rb: (rb, 0, 0)),
          pl.BlockSpec((D, D), lambda rb: (0, 0)),
          pl.BlockSpec((1, D), lambda rb: (0, 0)),
      ],
      out_specs=pl.BlockSpec((N_GRAPHS, D), lambda rb: (0, 0)),
      out_shape=jax.ShapeDtypeStruct((N_GRAPHS, D), F32),
      scratch_shapes=[
          pltpu.VMEM((N_GRAPHS, D), F32),
          pltpu.VMEM((N_GRAPHS, D), F32),
      ],
  )(hfin, valid3, batch3, wclsT, bcls2)


def kernel(x, edge_index, edge_attr, batch, W_lin, b_lin, gg_weight, gru_wih,
           gru_whh, gru_bih, gru_bhh, W_cls, b_cls):
  i32 = jnp.int32
  src = edge_index[0].astype(i32)
  dst = edge_index[1].astype(i32)
  ea = edge_attr.astype(i32)

  counts = jnp.bincount(ea, length=N_TYPES).astype(i32)
  nck = (counts + (CHUNK - 1)) // CHUNK
  cstart = jnp.concatenate([jnp.zeros((1,), i32), jnp.cumsum(nck)[:-1]])
  estart = jnp.concatenate([jnp.zeros((1,), i32), jnp.cumsum(counts)[:-1]])
  order = jnp.argsort(ea, stable=True)
  ta = ea[order]
  slot = cstart[ta] * CHUNK + (jnp.arange(N_EDGES, dtype=i32) - estart[ta])
  g_pad = jnp.zeros((C_TOT * CHUNK,), i32).at[slot].set(
      ta * N_PAD + src[order])
  d_pad = jnp.full((C_TOT * CHUNK,), TRASH, i32).at[slot].set(dst[order])
  ed = jnp.stack(
      [g_pad.reshape(C_TOT, CHUNK), d_pad.reshape(C_TOT, CHUNK)], axis=1)
  meta = jnp.zeros((32, 16), i32).at[:2 * N_TYPES].set(
      jnp.broadcast_to(
          jnp.stack([cstart, nck], axis=1).reshape(-1, 1), (2 * N_TYPES, 16)))

  valid3 = jnp.broadcast_to(
      (counts > 0).astype(F32)[:, None, None], (N_TYPES, 1, D))
  xpad = jnp.pad(x, ((0, N_PAD - N_NODES), (0, 0)))
  batch3 = jnp.pad(batch.astype(i32), (0, N_PAD - N_NODES),
                   constant_values=N_GRAPHS + 1).reshape(NRB, 1, RB)
  wlinT = W_lin.T
  bl2 = b_lin[None, :]
  wihT = jnp.transpose(gru_wih, (0, 2, 1))
  whhT = jnp.transpose(gru_whh, (0, 2, 1))
  bih3 = gru_bih[:, None, :]
  bhh3 = gru_bhh[:, None, :]
  wclsT = jnp.zeros((D, D), F32).at[:, :2].set(W_cls.T)
  bcls2 = jnp.zeros((1, D), F32).at[0, :2].set(b_cls)
  zeros128 = jnp.zeros((CHUNK, D), F32)

  sc_scatter = _make_sc_scatter()

  h, m = _init_call(xpad, wlinT, bl2, gg_weight[:, 0])
  for i in range(N_STEPS - 1):
    aggflat = sc_scatter(m.reshape(N_TYPES * N_PAD, D), ed, meta, zeros128)
    agg = aggflat.reshape(N_TYPES, N_PAD, D)
    h, m = _step_call(agg, h, wihT, whhT, bih3, bhh3, gg_weight[:, i + 1])
  aggflat = sc_scatter(m.reshape(N_TYPES * N_PAD, D), ed, meta, zeros128)
  agg = aggflat.reshape(N_TYPES, N_PAD, D)
  h = _last_step_call(agg, h, wihT, whhT, bih3, bhh3)

  out128 = _pool_call(h, valid3, batch3, wclsT, bcls2)
  return out128[:N_GRAPHS, :2]


# speedup vs baseline: 11.9447x; 1.0902x over previous
import functools

import jax
import jax.numpy as jnp
from jax import lax
from jax.experimental import pallas as pl
from jax.experimental.pallas import tpu as pltpu
from jax.experimental.pallas import tpu_sc as plsc

N_NODES = 10000
N_PAD = 10240
D = 128
N_TYPES = 13
N_STEPS = 8
N_GRAPHS = 64
N_EDGES = 320000
CHUNK = 128
C_TOT = N_EDGES // CHUNK + N_TYPES
RB = 512
NRB = N_PAD // RB
TRASH = N_NODES
ROWS_PER_TILE = N_PAD // 16
F32 = jnp.float32


GROUP = 2


def _sc_scatter_body(mflat, ed, meta, zeros, out, agg_sh, idx_v, rows_v,
                     meta_vm, sem_i, sem_g, sem_s):
  core = lax.axis_index("c")
  sub = lax.axis_index("s")
  pltpu.sync_copy(meta, meta_vm)

  def _idx_copy(slot, row):
    return pltpu.make_async_copy(ed.at[pl.ds(row, GROUP)], idx_v.at[slot],
                                 sem_i)

  def _gather_copy(slot, q):
    return pltpu.make_async_copy(mflat.at[idx_v.at[slot, q, 0]],
                                 rows_v.at[q], sem_g)

  def _scatter_copy(slot, q):
    return pltpu.make_async_copy(rows_v.at[q], agg_sh.at[idx_v.at[slot, q, 1]],
                                 sem_s)

  for t in range(N_TYPES):
    @pl.when(core == (t % 2))
    def _process_type():
      def _zero_copy(z):
        return pltpu.make_async_copy(
            zeros, agg_sh.at[pl.ds(sub * ROWS_PER_TILE + z * CHUNK, CHUNK)],
            sem_i)
      for z in range(ROWS_PER_TILE // CHUNK):
        _zero_copy(z).start()
      for z in range(ROWS_PER_TILE // CHUNK):
        _zero_copy(z).wait()
      plsc.subcore_barrier()
      start = jnp.max(meta_vm[2 * t])
      nck = jnp.max(meta_vm[2 * t + 1])
      quota = (nck + 15) // 16
      lo = sub * quota
      hi = jnp.minimum(nck, lo + quota)
      ngroups = (quota + GROUP - 1) // GROUP

      @pl.when(quota > 0)
      def _span():
        _idx_copy(0, start + lo).start()

        @pl.loop(0, ngroups)
        def _group(k):
          slot = lax.rem(k, 2)
          nslot = lax.rem(k + 1, 2)
          _idx_copy(slot, start + lo + k * GROUP).wait()
          _idx_copy(nslot, start + lo + (k + 1) * GROUP).start()
          for q in range(GROUP):
            @pl.when(lo + k * GROUP + q < hi)
            def _():
              _gather_copy(slot, q).start()
          for q in range(GROUP):
            @pl.when(lo + k * GROUP + q < hi)
            def _():
              _gather_copy(slot, q).wait()
          for q in range(GROUP):
            @pl.when(lo + k * GROUP + q < hi)
            def _():
              _scatter_copy(slot, q).start(add=True)
          for q in range(GROUP):
            @pl.when(lo + k * GROUP + q < hi)
            def _():
              _scatter_copy(slot, q).wait()

        _idx_copy(lax.rem(ngroups, 2), start + lo + ngroups * GROUP).wait()

      plsc.subcore_barrier()
      pltpu.sync_copy(
          agg_sh.at[pl.ds(sub * ROWS_PER_TILE, ROWS_PER_TILE)],
          out.at[pl.ds(t * N_PAD + sub * ROWS_PER_TILE, ROWS_PER_TILE)])


def _make_sc_scatter():
  mesh = plsc.VectorSubcoreMesh(core_axis_name="c", subcore_axis_name="s")
  return pl.kernel(
      _sc_scatter_body,
      out_type=jax.ShapeDtypeStruct((N_TYPES * N_PAD, D), F32),
      mesh=mesh,
      scratch_types=[
          pltpu.VMEM_SHARED((N_PAD, D), F32),
          pltpu.VMEM((2, GROUP, 2, CHUNK), jnp.int32),
          pltpu.VMEM((GROUP, CHUNK, D), F32),
          pltpu.VMEM((32, 16), jnp.int32),
          pltpu.SemaphoreType.DMA,
          pltpu.SemaphoreType.DMA,
          pltpu.SemaphoreType.DMA,
      ],
      compiler_params=pltpu.CompilerParams(needs_layout_passes=False),
  )


def _init_body(x_ref, wlt_ref, bl_ref, w0_ref, h_ref, m_ref):
  h0 = jnp.dot(x_ref[...], wlt_ref[...],
               preferred_element_type=F32) + bl_ref[...]
  h_ref[0] = h0
  m_ref[0] = jnp.dot(h0, w0_ref[0], preferred_element_type=F32)


def _gru_core(agg_ref, h_ref, wih_ref, whh_ref, bih_ref, bhh_ref):
  h = h_ref[0]
  gi = jnp.dot(agg_ref[0], wih_ref[0], preferred_element_type=F32) + bih_ref[0]
  gh = jnp.dot(h, whh_ref[0], preferred_element_type=F32) + bhh_ref[0]
  r = jax.nn.sigmoid(gi[:, :D] + gh[:, :D])
  z = jax.nn.sigmoid(gi[:, D:2 * D] + gh[:, D:2 * D])
  n = jnp.tanh(gi[:, 2 * D:] + r * gh[:, 2 * D:])
  return (1.0 - z) * n + z * h


def _step_body(agg_ref, h_ref, wih_ref, whh_ref, bih_ref, bhh_ref, wn_ref,
               hn_ref, mn_ref):
  hn = _gru_core(agg_ref, h_ref, wih_ref, whh_ref, bih_ref, bhh_ref)
  hn_ref[0] = hn
  mn_ref[0] = jnp.dot(hn, wn_ref[0], preferred_element_type=F32)


def _last_step_body(agg_ref, h_ref, wih_ref, whh_ref, bih_ref, bhh_ref,
                    hn_ref):
  hn_ref[0] = _gru_core(agg_ref, h_ref, wih_ref, whh_ref, bih_ref, bhh_ref)


def _pool_body(h_ref, valid_ref, batch_ref, wcls_ref, bcls_ref, out_ref,
               sums_ref, cnts_ref):
  rb = pl.program_id(0)

  @pl.when(rb == 0)
  def _():
    sums_ref[...] = jnp.zeros_like(sums_ref)
    cnts_ref[...] = jnp.zeros_like(cnts_ref)

  msg = jnp.zeros((RB, D), F32)
  for t in range(N_TYPES):
    msg = msg + h_ref[t] * valid_ref[t]
  b = batch_ref[0]
  gid = lax.broadcasted_iota(jnp.int32, (N_GRAPHS, RB), 0)
  onehot = (gid == jnp.broadcast_to(b, (N_GRAPHS, RB))).astype(F32)
  sums_ref[...] += jnp.dot(onehot, msg, preferred_element_type=F32)
  cnts_ref[...] += jnp.dot(onehot, jnp.ones((RB, D), F32),
                           preferred_element_type=F32)

  @pl.when(rb == NRB - 1)
  def _():
    pooled = sums_ref[...] / jnp.maximum(cnts_ref[...], 1.0)
    out_ref[...] = jnp.dot(pooled, wcls_ref[...],
                           preferred_element_type=F32) + bcls_ref[...]


def _tc_specs():
  blk_h = pl.BlockSpec((1, RB, D), lambda t, rb: (t, rb, 0))
  blk_w = pl.BlockSpec((1, D, 3 * D), lambda t, rb: (t, 0, 0))
  blk_b = pl.BlockSpec((1, 1, 3 * D), lambda t, rb: (t, 0, 0))
  blk_wn = pl.BlockSpec((1, D, D), lambda t, rb: (t, 0, 0))
  return blk_h, blk_w, blk_b, blk_wn


def _init_call(xpad, wlinT, bl2, w0):
  blk_h, _, _, blk_wn = _tc_specs()
  return pl.pallas_call(
      _init_body,
      grid=(N_TYPES, NRB),
      in_specs=[
          pl.BlockSpec((RB, D), lambda t, rb: (rb, 0)),
          pl.BlockSpec((D, D), lambda t, rb: (0, 0)),
          pl.BlockSpec((1, D), lambda t, rb: (0, 0)),
          blk_wn,
      ],
      out_specs=[blk_h, blk_h],
      out_shape=[
          jax.ShapeDtypeStruct((N_TYPES, N_PAD, D), F32),
          jax.ShapeDtypeStruct((N_TYPES, N_PAD, D), F32),
      ],
  )(xpad, wlinT, bl2, w0)


def _step_call(agg, h, wihT, whhT, bih3, bhh3, wn):
  blk_h, blk_w, blk_b, blk_wn = _tc_specs()
  return pl.pallas_call(
      _step_body,
      grid=(N_TYPES, NRB),
      in_specs=[blk_h, blk_h, blk_w, blk_w, blk_b, blk_b, blk_wn],
      out_specs=[blk_h, blk_h],
      out_shape=[
          jax.ShapeDtypeStruct((N_TYPES, N_PAD, D), F32),
          jax.ShapeDtypeStruct((N_TYPES, N_PAD, D), F32),
      ],
  )(agg, h, wihT, whhT, bih3, bhh3, wn)


def _last_step_call(agg, h, wihT, whhT, bih3, bhh3):
  blk_h, blk_w, blk_b, _ = _tc_specs()
  return pl.pallas_call(
      _last_step_body,
      grid=(N_TYPES, NRB),
      in_specs=[blk_h, blk_h, blk_w, blk_w, blk_b, blk_b],
      out_specs=blk_h,
      out_shape=jax.ShapeDtypeStruct((N_TYPES, N_PAD, D), F32),
  )(agg, h, wihT, whhT, bih3, bhh3)


def _pool_call(hfin, valid3, batch3, wclsT, bcls2):
  return pl.pallas_call(
      _pool_body,
      grid=(NRB,),
      in_specs=[
          pl.BlockSpec((N_TYPES, RB, D), lambda rb: (0, rb, 0)),
          pl.BlockSpec((N_TYPES, 1, D), lambda rb: (0, 0, 0)),
          pl.BlockSpec((1, 1, RB), lambda rb: (rb, 0, 0)),
          pl.BlockSpec((D, D), lambda rb: (0, 0)),
          pl.BlockSpec((1, D), lambda rb: (0, 0)),
      ],
      out_specs=pl.BlockSpec((N_GRAPHS, D), lambda rb: (0, 0)),
      out_shape=jax.ShapeDtypeStruct((N_GRAPHS, D), F32),
      scratch_shapes=[
          pltpu.VMEM((N_GRAPHS, D), F32),
          pltpu.VMEM((N_GRAPHS, D), F32),
      ],
  )(hfin, valid3, batch3, wclsT, bcls2)


def kernel(x, edge_index, edge_attr, batch, W_lin, b_lin, gg_weight, gru_wih,
           gru_whh, gru_bih, gru_bhh, W_cls, b_cls):
  i32 = jnp.int32
  src = edge_index[0].astype(i32)
  dst = edge_index[1].astype(i32)
  ea = edge_attr.astype(i32)

  counts = jnp.bincount(ea, length=N_TYPES).astype(i32)
  nck = (counts + (CHUNK - 1)) // CHUNK
  cstart = jnp.concatenate([jnp.zeros((1,), i32), jnp.cumsum(nck)[:-1]])
  estart = jnp.concatenate([jnp.zeros((1,), i32), jnp.cumsum(counts)[:-1]])
  order = jnp.argsort(ea, stable=True)
  ta = ea[order]
  slot = cstart[ta] * CHUNK + (jnp.arange(N_EDGES, dtype=i32) - estart[ta])
  c_alloc = C_TOT + 32
  g_pad = jnp.zeros((c_alloc * CHUNK,), i32).at[slot].set(
      ta * N_PAD + src[order])
  d_pad = jnp.full((c_alloc * CHUNK,), TRASH, i32).at[slot].set(dst[order])
  ed = jnp.stack(
      [g_pad.reshape(c_alloc, CHUNK), d_pad.reshape(c_alloc, CHUNK)], axis=1)
  meta = jnp.zeros((32, 16), i32).at[:2 * N_TYPES].set(
      jnp.broadcast_to(
          jnp.stack([cstart, nck], axis=1).reshape(-1, 1), (2 * N_TYPES, 16)))

  valid3 = jnp.broadcast_to(
      (counts > 0).astype(F32)[:, None, None], (N_TYPES, 1, D))
  xpad = jnp.pad(x, ((0, N_PAD - N_NODES), (0, 0)))
  batch3 = jnp.pad(batch.astype(i32), (0, N_PAD - N_NODES),
                   constant_values=N_GRAPHS + 1).reshape(NRB, 1, RB)
  wlinT = W_lin.T
  bl2 = b_lin[None, :]
  wihT = jnp.transpose(gru_wih, (0, 2, 1))
  whhT = jnp.transpose(gru_whh, (0, 2, 1))
  bih3 = gru_bih[:, None, :]
  bhh3 = gru_bhh[:, None, :]
  wclsT = jnp.zeros((D, D), F32).at[:, :2].set(W_cls.T)
  bcls2 = jnp.zeros((1, D), F32).at[0, :2].set(b_cls)
  zeros128 = jnp.zeros((CHUNK, D), F32)

  sc_scatter = _make_sc_scatter()

  h, m = _init_call(xpad, wlinT, bl2, gg_weight[:, 0])
  for i in range(N_STEPS - 1):
    aggflat = sc_scatter(m.reshape(N_TYPES * N_PAD, D), ed, meta, zeros128)
    agg = aggflat.reshape(N_TYPES, N_PAD, D)
    h, m = _step_call(agg, h, wihT, whhT, bih3, bhh3, gg_weight[:, i + 1])
  aggflat = sc_scatter(m.reshape(N_TYPES * N_PAD, D), ed, meta, zeros128)
  agg = aggflat.reshape(N_TYPES, N_PAD, D)
  h = _last_step_call(agg, h, wihT, whhT, bih3, bhh3)

  out128 = _pool_call(h, valid3, batch3, wclsT, bcls2)
  return out128[:N_GRAPHS, :2]
